# baseline (speedup 1.0000x reference)
import jax
import jax.numpy as jnp
from jax.experimental import pallas as pl
from jax.experimental.pallas import tpu as pltpu

K, B, N, D = 6, 64, 100000, 128
TNo = 11136
SUB = 3
TNr = TNo * SUB
NR = 3
NBo = NR * SUB
LAST_OFF = N - TNr
SHIFT = (NR - 1) * TNr - LAST_OFF
REMC = N - (NBo - 1) * TNo
C1 = TNo


def _sim_body(pf_ref, mem_ref, out_ref, f16_ref, in_bufs, rsems, psems):
    k = pl.program_id(0)
    n = pl.program_id(1)
    jl = n // SUB
    s = n % SUB
    j = k * NR + jl
    slot = j % 2

    @pl.when(n == 0)
    def _():
        f = pf_ref[0]
        norm = jnp.sqrt(jnp.sum(f * f, axis=1, keepdims=True))
        f16_ref[...] = (f / jnp.maximum(norm, 1e-12)).astype(jnp.bfloat16)

    def read_copy(jk, jn, dst_slot):
        off = jnp.minimum(jn * TNr, N - TNr)
        return pltpu.make_async_copy(
            mem_ref.at[jk, pl.ds(off, TNr), :],
            in_bufs.at[dst_slot],
            rsems.at[dst_slot],
        )

    first = (k == 0) & (n == 0)

    @pl.when(first)
    def _():
        pltpu.make_async_copy(
            mem_ref.at[0, pl.ds(0, C1), :],
            in_bufs.at[0, pl.ds(0, C1), :],
            psems.at[0],
        ).start()
        pltpu.make_async_copy(
            mem_ref.at[0, pl.ds(C1, TNr - C1), :],
            in_bufs.at[0, pl.ds(C1, TNr - C1), :],
            psems.at[1],
        ).start()
        read_copy(0, 1, 1).start()

    @pl.when((s == 0) & jnp.logical_not(first) & (j + 1 < K * NR))
    def _():
        nj = j + 1
        read_copy(nj // NR, nj % NR, nj % 2).start()

    @pl.when(first)
    def _():
        pltpu.make_async_copy(
            mem_ref.at[0, pl.ds(0, C1), :],
            in_bufs.at[0, pl.ds(0, C1), :],
            psems.at[0],
        ).wait()

    @pl.when((k == 0) & (n == 1))
    def _():
        pltpu.make_async_copy(
            mem_ref.at[0, pl.ds(C1, TNr - C1), :],
            in_bufs.at[0, pl.ds(C1, TNr - C1), :],
            psems.at[1],
        ).wait()

    @pl.when((s == 0) & jnp.logical_not(first) & (j > 0))
    def _():
        read_copy(k, jl, slot).wait()

    loc = s * TNo + jnp.where(jl == NR - 1, SHIFT, 0)

    @pl.when(n < NBo - 1)
    def _():
        m = in_bufs[slot, pl.ds(loc, TNo), :]
        out_ref[0] = jax.lax.dot_general(
            f16_ref[...], m.astype(jnp.bfloat16),
            (((1,), (1,)), ((), ())), preferred_element_type=jnp.float32,
        )

    @pl.when(n == NBo - 1)
    def _():
        m = in_bufs[slot, pl.ds(loc, REMC), :]
        out_ref[0, :, :REMC] = jax.lax.dot_general(
            f16_ref[...], m.astype(jnp.bfloat16),
            (((1,), (1,)), ((), ())), preferred_element_type=jnp.float32,
        )


def kernel(part_features, memory):
    return pl.pallas_call(
        _sim_body,
        grid=(K, NBo),
        in_specs=[
            pl.BlockSpec((1, B, D), lambda k, n: (k, 0, 0)),
            pl.BlockSpec(memory_space=pl.ANY),
        ],
        out_specs=pl.BlockSpec((1, B, TNo), lambda k, n: (k, 0, n)),
        out_shape=jax.ShapeDtypeStruct((K, B, N), jnp.float32),
        scratch_shapes=[
            pltpu.VMEM((B, D), jnp.bfloat16),
            pltpu.VMEM((2, TNr, D), jnp.float32),
            pltpu.SemaphoreType.DMA((2,)),
            pltpu.SemaphoreType.DMA((2,)),
        ],
        compiler_params=pltpu.CompilerParams(
            dimension_semantics=("arbitrary", "arbitrary"),
        ),
    )(part_features, memory)


# speedup vs baseline: 1.0389x; 1.0389x over previous
import jax
import jax.numpy as jnp
from jax.experimental import pallas as pl
from jax.experimental.pallas import tpu as pltpu

K, B, N, D = 6, 64, 100000, 128
TN = 33408


def _sim_body(pf_ref, mem_ref, out_ref, f16_ref):
    n = pl.program_id(1)

    @pl.when(n == 0)
    def _():
        f = pf_ref[0]
        norm = jnp.sqrt(jnp.sum(f * f, axis=1, keepdims=True))
        f16_ref[...] = (f / jnp.maximum(norm, 1e-12)).astype(jnp.bfloat16)

    m = mem_ref[0].astype(jnp.bfloat16)
    out_ref[0] = jax.lax.dot_general(
        f16_ref[...], m, (((1,), (1,)), ((), ())),
        preferred_element_type=jnp.float32,
    )


def kernel(part_features, memory):
    nb = pl.cdiv(N, TN)
    return pl.pallas_call(
        _sim_body,
        grid=(K, nb),
        in_specs=[
            pl.BlockSpec((1, B, D), lambda k, n: (k, 0, 0)),
            pl.BlockSpec((1, TN, D), lambda k, n: (k, n, 0)),
        ],
        out_specs=pl.BlockSpec((1, B, TN), lambda k, n: (k, 0, n)),
        out_shape=jax.ShapeDtypeStruct((K, B, N), jnp.float32),
        scratch_shapes=[pltpu.VMEM((B, D), jnp.bfloat16)],
        compiler_params=pltpu.CompilerParams(
            dimension_semantics=("parallel", "arbitrary"),
        ),
    )(part_features, memory)
